# Initial kernel scaffold; baseline (speedup 1.0000x reference)
import functools

import jax
import jax.numpy as jnp
from jax import lax
from jax.experimental import pallas as pl
from jax.experimental.pallas import tpu as pltpu
from jax.experimental.pallas import tpu_sc as plsc

N = 10000
D = 128
NNZ = 320000
NC = 2
NS = 16
NW = NC * NS
K = 128
CH = (NNZ + NW * K - 1) // (NW * K)
PER_W = CH * K
PAD = PER_W * NW
RPS = N // NS
CW = 16


def _sc_phase(table, gidx, sidx, z128, z16, ones):
    mesh = plsc.VectorSubcoreMesh(core_axis_name="c", subcore_axis_name="s")

    @functools.partial(
        pl.kernel,
        out_type=[
            jax.ShapeDtypeStruct((NC, N, D), jnp.float32),
            jax.ShapeDtypeStruct((NC, N, CW), jnp.float32),
        ],
        mesh=mesh,
        scratch_types=[
            pltpu.VMEM((CH, K), jnp.int32),
            pltpu.VMEM((CH, K), jnp.int32),
            pltpu.VMEM((K, D), jnp.float32),
            pltpu.VMEM((K, CW), jnp.float32),
            pltpu.VMEM_SHARED((N + 1, D), jnp.float32),
            pltpu.VMEM_SHARED((N + 1, CW), jnp.float32),
            pltpu.SemaphoreType.DMA,
        ],
    )
    def phase(tab_hbm, gidx_hbm, sidx_hbm, z128_hbm, z16_hbm, ones_hbm,
              acc_out, cnt_out, gi_v, si_v, rows_v, ones_v, acc_sp, cnt_sp,
              sem):
        c = lax.axis_index("c")
        s = lax.axis_index("s")
        w = s * NC + c
        base = s * RPS
        pltpu.sync_copy(gidx_hbm.at[w], gi_v)
        pltpu.sync_copy(sidx_hbm.at[w], si_v)
        pltpu.sync_copy(ones_hbm, ones_v)
        pltpu.sync_copy(z128_hbm, acc_sp.at[pl.ds(base, RPS)])
        pltpu.sync_copy(z16_hbm, cnt_sp.at[pl.ds(base, RPS)])
        plsc.subcore_barrier()

        def step(j, carry):
            pltpu.async_copy(tab_hbm.at[gi_v.at[j]], rows_v, sem).wait()
            pltpu.sync_copy(rows_v, acc_sp.at[si_v.at[j]], add=True)
            pltpu.sync_copy(ones_v, cnt_sp.at[si_v.at[j]], add=True)
            return carry

        lax.fori_loop(0, CH, step, 0)
        plsc.subcore_barrier()
        pltpu.sync_copy(acc_sp.at[pl.ds(base, RPS)],
                        acc_out.at[c, pl.ds(base, RPS)])
        pltpu.sync_copy(cnt_sp.at[pl.ds(base, RPS)],
                        cnt_out.at[c, pl.ds(base, RPS)])

    return phase(table, gidx, sidx, z128, z16, ones)


def _combine(acc_p, cnt_p, bias2d, final):
    BR = 625

    def body(a_ref, c_ref, b_ref, o_ref):
        sums = a_ref[0] + a_ref[1]
        cnt = c_ref[0, :, 0:1] + c_ref[1, :, 0:1]
        recip = jnp.where(cnt == 0.0, 0.0,
                          1.0 / jnp.where(cnt == 0.0, 1.0, cnt))
        y = sums * recip
        if final:
            y = y + b_ref[0]
            y = jnp.where(y >= 0.0, y, 0.01 * y)
        o_ref[...] = y

    return pl.pallas_call(
        body,
        grid=(N // BR,),
        in_specs=[
            pl.BlockSpec((2, BR, D), lambda i: (0, i, 0)),
            pl.BlockSpec((2, BR, CW), lambda i: (0, i, 0)),
            pl.BlockSpec((1, D), lambda i: (0, 0)),
        ],
        out_specs=pl.BlockSpec((BR, D), lambda i: (i, 0)),
        out_shape=jax.ShapeDtypeStruct((N, D), jnp.float32),
    )(acc_p, cnt_p, bias2d)


def kernel(x, hyperedge_index, bias):
    src = hyperedge_index[0].astype(jnp.int32)
    dst = hyperedge_index[1].astype(jnp.int32)
    npad = PAD - NNZ
    pad_s = jnp.full((npad,), N, jnp.int32)
    pad_g = jnp.zeros((npad,), jnp.int32)
    src_g = jnp.concatenate([src, pad_g]).reshape(NW, CH, K)
    src_s = jnp.concatenate([src, pad_s]).reshape(NW, CH, K)
    dst_g = jnp.concatenate([dst, pad_g]).reshape(NW, CH, K)
    dst_s = jnp.concatenate([dst, pad_s]).reshape(NW, CH, K)

    z128 = jnp.zeros((RPS, D), jnp.float32)
    z16 = jnp.zeros((RPS, CW), jnp.float32)
    ones = jnp.ones((K, CW), jnp.float32)
    bias2d = bias.reshape(1, D)

    acc1, cnt_dst = _sc_phase(x, src_g, dst_s, z128, z16, ones)
    out_e = _combine(acc1, cnt_dst, bias2d, final=False)
    acc2, cnt_src = _sc_phase(out_e, dst_g, src_s, z128, z16, ones)
    return _combine(acc2, cnt_src, bias2d, final=True)


# speedup vs baseline: 8.2829x; 8.2829x over previous
import functools

import jax
import jax.numpy as jnp
from jax import lax
from jax.experimental import pallas as pl
from jax.experimental.pallas import tpu as pltpu
from jax.experimental.pallas import tpu_sc as plsc

N = 10000
D = 128
NNZ = 320000
NC = 2
NS = 16
NW = NC * NS
K = 128
G = 8
CH = -(-NNZ // (NW * K * G)) * G
PER_W = CH * K
PAD = PER_W * NW
RPS = 632
NP = RPS * NS
HR = NP // 128
L = 16


def _sc_phase(table, gidx, sidx, z128):
    mesh = plsc.VectorSubcoreMesh(core_axis_name="c", subcore_axis_name="s")

    @functools.partial(
        pl.kernel,
        out_type=[
            jax.ShapeDtypeStruct((NC, NP, D), jnp.float32),
            jax.ShapeDtypeStruct((NW, HR, 128), jnp.float32),
        ],
        mesh=mesh,
        scratch_types=[
            pltpu.VMEM((G, K), jnp.int32),
            pltpu.VMEM((G, K), jnp.int32),
            pltpu.VMEM((K, D), jnp.float32),
            pltpu.VMEM((HR, 128), jnp.float32),
            pltpu.VMEM_SHARED((NP + 8, D), jnp.float32),
            pltpu.SemaphoreType.DMA,
        ],
        compiler_params=pltpu.CompilerParams(needs_layout_passes=False),
    )
    def phase(tab_hbm, gidx_hbm, sidx_hbm, z128_hbm,
              acc_out, hist_out, gi_v, si_v, rows_v, hist_v, acc_sp, sem):
        c = lax.axis_index("c")
        s = lax.axis_index("s")
        w = s * NC + c
        base = s * RPS
        pltpu.sync_copy(z128_hbm, acc_sp.at[pl.ds(base, RPS)])

        zv = jnp.zeros((L,), jnp.float32)

        def zrow(r, carry):
            for l in range(128 // L):
                hist_v[r, pl.ds(l * L, L)] = zv
            return carry

        lax.fori_loop(0, HR, zrow, 0)
        plsc.subcore_barrier()

        def group(g, carry):
            pltpu.sync_copy(gidx_hbm.at[w, pl.ds(g * G, G)], gi_v)
            pltpu.sync_copy(sidx_hbm.at[w, pl.ds(g * G, G)], si_v)
            for j in range(G):
                pltpu.async_copy(tab_hbm.at[gi_v.at[j]], rows_v, sem).wait()
                pltpu.sync_copy(rows_v, acc_sp.at[si_v.at[j]], add=True)
                for u in range(K // L):
                    v = si_v[j, pl.ds(u * L, L)]
                    cnt, last = plsc.scan_count(v)
                    r = lax.shift_right_logical(v, 7)
                    col = lax.bitwise_and(v, 127)
                    plsc.addupdate_scatter(
                        hist_v, [r, col], cnt.astype(jnp.float32), mask=last)
            return carry

        lax.fori_loop(0, CH // G, group, 0)
        plsc.subcore_barrier()
        pltpu.sync_copy(acc_sp.at[pl.ds(base, RPS)],
                        acc_out.at[c, pl.ds(base, RPS)])
        pltpu.sync_copy(hist_v, hist_out.at[w])

    return phase(table, gidx, sidx, z128)


def _combine(acc_p, hist_p, bias2d, final):
    def body(a_ref, h_ref, b_ref, o_ref):
        sums = a_ref[0] + a_ref[1]
        cnt = jnp.sum(h_ref[0], axis=0)
        rec = jnp.where(cnt == 0.0, 0.0,
                        1.0 / jnp.where(cnt == 0.0, 1.0, cnt))
        ri = lax.broadcasted_iota(jnp.int32, (128, 128), 0)
        ci = lax.broadcasted_iota(jnp.int32, (128, 128), 1)
        dg = jnp.where(ri == ci, rec[None, :], 0.0)
        y = jax.lax.dot(dg, sums,
                        precision=jax.lax.Precision.HIGHEST,
                        preferred_element_type=jnp.float32)
        if final:
            y = y + b_ref[...]
            y = jnp.where(y >= 0.0, y, 0.01 * y)
        o_ref[...] = y

    return pl.pallas_call(
        body,
        grid=(HR,),
        in_specs=[
            pl.BlockSpec((NC, 128, D), lambda i: (0, i, 0)),
            pl.BlockSpec((1, NW, 128), lambda i: (i, 0, 0)),
            pl.BlockSpec((1, D), lambda i: (0, 0)),
        ],
        out_specs=pl.BlockSpec((128, D), lambda i: (i, 0)),
        out_shape=jax.ShapeDtypeStruct((NP, D), jnp.float32),
    )(acc_p, hist_p, bias2d)


def kernel(x, hyperedge_index, bias):
    src = hyperedge_index[0].astype(jnp.int32)
    dst = hyperedge_index[1].astype(jnp.int32)
    npad = PAD - NNZ
    pad_s = jnp.full((npad,), N, jnp.int32)
    pad_g = jnp.zeros((npad,), jnp.int32)
    src_g = jnp.concatenate([src, pad_g]).reshape(NW, CH, K)
    src_s = jnp.concatenate([src, pad_s]).reshape(NW, CH, K)
    dst_g = jnp.concatenate([dst, pad_g]).reshape(NW, CH, K)
    dst_s = jnp.concatenate([dst, pad_s]).reshape(NW, CH, K)

    z128 = jnp.zeros((RPS, D), jnp.float32)
    bias2d = bias.reshape(1, D)

    acc1, hist_dst = _sc_phase(x, src_g, dst_s, z128)
    out_e = _combine(acc1, hist_dst.transpose(1, 0, 2), bias2d, final=False)
    acc2, hist_src = _sc_phase(out_e, dst_g, src_s, z128)
    out = _combine(acc2, hist_src.transpose(1, 0, 2), bias2d, final=True)
    return out[:N]
